# baseline (speedup 1.0000x reference)
import functools

import jax
import jax.numpy as jnp
from jax import lax
from jax.experimental import pallas as pl
from jax.experimental.pallas import tpu as pltpu
from jax.experimental.pallas import tpu_sc as plsc

N = 10000
D = 128
E = 320000
CB = 80
NC, NS = 2, 16
NW = NC * NS
CHUNKS_PER_TILE = E // (NW * CB)
GROUPS = 5
CPG = CHUNKS_PER_TILE // GROUPS
NP = 10240
ROWS_PER_TILE = NP // NS
ZR = 16
NB = 3
LA = NB - 1
R = 1000


@functools.cache
def _make_sc_agg(with_counts):
    out_type = [jax.ShapeDtypeStruct((NC, NP, D), jnp.float32)]
    scratch = [
        pltpu.VMEM_SHARED((NP, D), jnp.float32),
        pltpu.VMEM((CPG, CB), jnp.int32),
        pltpu.VMEM((CPG, CB), jnp.int32),
        pltpu.VMEM((NB, CB, D), jnp.float32),
        pltpu.SemaphoreType.DMA,
        pltpu.SemaphoreType.DMA,
    ]
    if with_counts:
        out_type.append(jax.ShapeDtypeStruct((NC, NP), jnp.float32))
        scratch += [
            pltpu.VMEM_SHARED((NP,), jnp.float32),
            pltpu.VMEM((CB,), jnp.float32),
            pltpu.SemaphoreType.DMA,
        ]
    mesh = plsc.VectorSubcoreMesh(core_axis_name="c", subcore_axis_name="s")

    @functools.partial(
        pl.kernel,
        out_type=out_type,
        mesh=mesh,
        scratch_types=scratch,
        compiler_params=pltpu.CompilerParams(use_tc_tiling_on_sc=False),
    )
    def agg(table_hbm, edge_hbm, zero_hbm, *rest):
        if with_counts:
            zero1_hbm, out_hbm, cnt_hbm, acc_sh, idx_src, idx_dst, rows, \
                gsem, ssem, cnt_sh, vec1, csem = rest
        else:
            out_hbm, acc_sh, idx_src, idx_dst, rows, gsem, ssem = rest
        c = lax.axis_index("c")
        s = lax.axis_index("s")

        zacc = pltpu.async_copy(
            zero_hbm, acc_sh.at[pl.ds(s * ROWS_PER_TILE, ROWS_PER_TILE), :],
            gsem)
        if with_counts:
            zcnt = pltpu.async_copy(
                zero1_hbm,
                cnt_sh.at[pl.ds(s * ROWS_PER_TILE, ROWS_PER_TILE)], csem)
            ovec = jnp.full((16,), 1.0, jnp.float32)

            def oc(r, carry):
                vec1[pl.ds(r * 16, 16)] = ovec
                return carry
            lax.fori_loop(0, CB // 16, oc, 0)
            zcnt.wait()
        zacc.wait()
        plsc.subcore_barrier()

        tile0 = (c * NS + s) * CHUNKS_PER_TILE

        def g_issue(j, b):
            pltpu.async_copy(table_hbm.at[idx_src.at[j]], rows.at[b], gsem)

        def g_wait(j, b):
            pltpu.make_async_copy(table_hbm.at[idx_src.at[j]], rows.at[b],
                                  gsem).wait()

        def s_issue(j, b):
            pltpu.async_copy(rows.at[b], acc_sh.at[idx_dst.at[j]], ssem,
                             add=True)
            if with_counts:
                pltpu.async_copy(vec1.at[pl.ds(0, CB)],
                                 cnt_sh.at[idx_dst.at[j]], csem, add=True)

        def s_drain():
            pltpu.make_async_copy(rows.at[0], acc_sh.at[idx_dst.at[0]],
                                  ssem).wait()

        def group(g, carry):
            row0 = tile0 + g * CPG
            pltpu.sync_copy(edge_hbm.at[0, pl.ds(row0, CPG), :], idx_src)
            pltpu.sync_copy(edge_hbm.at[1, pl.ds(row0, CPG), :], idx_dst)

            for j in range(LA):
                g_issue(j, j)
            g_wait(0, 0)
            s_issue(0, 0)
            g_issue(LA, LA)

            def chunk(j, carry2):
                b = j % NB
                g_wait(j, b)
                s_issue(j, b)
                s_drain()
                g_issue(j + LA, (j + LA) % NB)
                return carry2

            lax.fori_loop(1, CPG - LA, chunk, carry)

            for j in range(CPG - LA, CPG):
                b = j % NB
                g_wait(j, b)
                s_issue(j, b)
                s_drain()
            s_drain()

            if with_counts:
                def cdrain(j, carry2):
                    pltpu.make_async_copy(vec1.at[pl.ds(0, CB)],
                                          cnt_sh.at[idx_dst.at[0]],
                                          csem).wait()
                    return carry2
                lax.fori_loop(0, CPG, cdrain, carry)
            return carry

        lax.fori_loop(0, GROUPS, group, 0)
        plsc.subcore_barrier()

        pltpu.sync_copy(
            acc_sh.at[pl.ds(s * ROWS_PER_TILE, ROWS_PER_TILE), :],
            out_hbm.at[c, pl.ds(s * ROWS_PER_TILE, ROWS_PER_TILE), :])
        if with_counts:
            pltpu.sync_copy(
                cnt_sh.at[pl.ds(s * ROWS_PER_TILE, ROWS_PER_TILE)],
                cnt_hbm.at[c, pl.ds(s * ROWS_PER_TILE, ROWS_PER_TILE)])

    return agg


_DN = (((1,), (1,)), ((), ()))


def _tc_self_body(x_ref, w_ref, b_ref, o_ref):
    o_ref[...] = lax.dot_general(
        x_ref[...], w_ref[...], _DN,
        preferred_element_type=jnp.float32) + b_ref[...]


def _tc_mix1_body(P_ref, C_ref, s_ref, wl_ref, h_ref, ic_ref):
    acc = P_ref[0] + P_ref[1]
    cnt = C_ref[0] + C_ref[1]
    inv = 1.0 / jnp.maximum(cnt, 1.0)
    h = lax.dot_general(acc * inv, wl_ref[...], _DN,
                        preferred_element_type=jnp.float32) + s_ref[...]
    h_ref[...] = jnp.maximum(h, 0.0)
    ic_ref[...] = inv


def _tc_mix2_body(P_ref, s_ref, ic_ref, wl_ref, out_ref):
    acc = P_ref[0] + P_ref[1]
    out_ref[...] = lax.dot_general(
        acc * ic_ref[...], wl_ref[...], _DN,
        preferred_element_type=jnp.float32) + s_ref[...]


def _tc_self(x, w, b):
    return pl.pallas_call(
        _tc_self_body,
        grid=(N // R,),
        in_specs=[
            pl.BlockSpec((R, D), lambda i: (i, 0)),
            pl.BlockSpec((D, D), lambda i: (0, 0)),
            pl.BlockSpec((1, D), lambda i: (0, 0)),
        ],
        out_specs=pl.BlockSpec((R, D), lambda i: (i, 0)),
        out_shape=jax.ShapeDtypeStruct((N, D), jnp.float32),
    )(x, w, b)


def _tc_mix1(P, C3, s, wl):
    return pl.pallas_call(
        _tc_mix1_body,
        grid=(N // R,),
        in_specs=[
            pl.BlockSpec((NC, R, D), lambda i: (0, i, 0)),
            pl.BlockSpec((NC, R, 1), lambda i: (0, i, 0)),
            pl.BlockSpec((R, D), lambda i: (i, 0)),
            pl.BlockSpec((D, D), lambda i: (0, 0)),
        ],
        out_specs=[
            pl.BlockSpec((R, D), lambda i: (i, 0)),
            pl.BlockSpec((R, 1), lambda i: (i, 0)),
        ],
        out_shape=[
            jax.ShapeDtypeStruct((N, D), jnp.float32),
            jax.ShapeDtypeStruct((N, 1), jnp.float32),
        ],
    )(P, C3, s, wl)


def _tc_mix2(P, s, ic, wl):
    return pl.pallas_call(
        _tc_mix2_body,
        grid=(N // R,),
        in_specs=[
            pl.BlockSpec((NC, R, D), lambda i: (0, i, 0)),
            pl.BlockSpec((R, D), lambda i: (i, 0)),
            pl.BlockSpec((R, 1), lambda i: (i, 0)),
            pl.BlockSpec((D, D), lambda i: (0, 0)),
        ],
        out_specs=pl.BlockSpec((R, D), lambda i: (i, 0)),
        out_shape=jax.ShapeDtypeStruct((N, D), jnp.float32),
    )(P, s, ic, wl)


def kernel(x, edge_index, W1_l, b1_l, W1_r, W2_l, b2_l, W2_r):
    edge3 = edge_index.reshape(2, E // CB, CB)
    z2 = jnp.zeros((ROWS_PER_TILE, D), jnp.float32)
    z1 = jnp.zeros((ROWS_PER_TILE,), jnp.float32)
    s1 = _tc_self(x, W1_r, b1_l.reshape(1, D))
    P1, C1 = _make_sc_agg(True)(x, edge3, z2, z1)
    h, ic = _tc_mix1(P1, C1.reshape(NC, NP, 1), s1, W1_l)
    s2 = _tc_self(h, W2_r, b2_l.reshape(1, D))
    P2, = _make_sc_agg(False)(h, edge3, z2)
    return _tc_mix2(P2, s2, ic, W2_l)


# speedup vs baseline: 1.0507x; 1.0507x over previous
import functools

import jax
import jax.numpy as jnp
from jax import lax
from jax.experimental import pallas as pl
from jax.experimental.pallas import tpu as pltpu
from jax.experimental.pallas import tpu_sc as plsc

N = 10000
D = 128
E = 320000
CB = 80
NC, NS = 2, 16
NW = NC * NS
CHUNKS_PER_TILE = E // (NW * CB)
GROUPS = 5
CPG = CHUNKS_PER_TILE // GROUPS
NP = 10240
ROWS_PER_TILE = NP // NS
ZR = 32
NB = 3
LA = NB - 1
R = 1000


@functools.cache
def _make_sc_agg(with_counts):
    out_type = [jax.ShapeDtypeStruct((NC, NP, D), jnp.float32)]
    scratch = [
        pltpu.VMEM_SHARED((NP, D), jnp.float32),
        pltpu.VMEM((CPG, CB), jnp.int32),
        pltpu.VMEM((CPG, CB), jnp.int32),
        pltpu.VMEM((NB, CB, D), jnp.float32),
        pltpu.VMEM((ZR, D), jnp.float32),
        pltpu.SemaphoreType.DMA,
        pltpu.SemaphoreType.DMA,
    ]
    if with_counts:
        out_type.append(jax.ShapeDtypeStruct((NC, NP), jnp.float32))
        scratch += [
            pltpu.VMEM_SHARED((NP,), jnp.float32),
            pltpu.VMEM((ROWS_PER_TILE,), jnp.float32),
            pltpu.SemaphoreType.DMA,
        ]
    mesh = plsc.VectorSubcoreMesh(core_axis_name="c", subcore_axis_name="s")

    @functools.partial(
        pl.kernel,
        out_type=out_type,
        mesh=mesh,
        scratch_types=scratch,
        compiler_params=pltpu.CompilerParams(use_tc_tiling_on_sc=False),
    )
    def agg(table_hbm, edge_hbm, out_hbm, *rest):
        if with_counts:
            cnt_hbm, acc_sh, idx_src, idx_dst, rows, zbuf, gsem, ssem, \
                cnt_sh, vec1, csem = rest
        else:
            acc_sh, idx_src, idx_dst, rows, zbuf, gsem, ssem = rest
        c = lax.axis_index("c")
        s = lax.axis_index("s")

        zvec = jnp.zeros((16,), jnp.float32)

        def zrow(r, carry):
            def zcol(q, carry2):
                zbuf[r, pl.ds(q * 16, 16)] = zvec
                return carry2
            return lax.fori_loop(0, D // 16, zcol, carry)

        lax.fori_loop(0, ZR, zrow, 0)
        for t in range(ROWS_PER_TILE // ZR):
            pltpu.async_copy(
                zbuf, acc_sh.at[pl.ds(s * ROWS_PER_TILE + t * ZR, ZR), :],
                ssem)
        if with_counts:
            def zc(r, carry):
                vec1[pl.ds(r * 16, 16)] = zvec
                return carry
            lax.fori_loop(0, ROWS_PER_TILE // 16, zc, 0)
            pltpu.sync_copy(vec1,
                            cnt_sh.at[pl.ds(s * ROWS_PER_TILE,
                                            ROWS_PER_TILE)])
        for t in range(ROWS_PER_TILE // ZR):
            pltpu.make_async_copy(
                zbuf, acc_sh.at[pl.ds(s * ROWS_PER_TILE + t * ZR, ZR), :],
                ssem).wait()
        plsc.subcore_barrier()
        if with_counts:
            ovec = jnp.full((16,), 1.0, jnp.float32)

            def oc(r, carry):
                vec1[pl.ds(r * 16, 16)] = ovec
                return carry
            lax.fori_loop(0, CB // 16, oc, 0)

        tile0 = (c * NS + s) * CHUNKS_PER_TILE

        def g_issue(j, b):
            pltpu.async_copy(table_hbm.at[idx_src.at[j]], rows.at[b], gsem)

        def g_wait(j, b):
            pltpu.make_async_copy(table_hbm.at[idx_src.at[j]], rows.at[b],
                                  gsem).wait()

        def s_issue(j, b):
            pltpu.async_copy(rows.at[b], acc_sh.at[idx_dst.at[j]], ssem,
                             add=True)
            if with_counts:
                pltpu.async_copy(vec1.at[pl.ds(0, CB)],
                                 cnt_sh.at[idx_dst.at[j]], csem, add=True)

        def s_drain():
            pltpu.make_async_copy(rows.at[0], acc_sh.at[idx_dst.at[0]],
                                  ssem).wait()

        def group(g, carry):
            row0 = tile0 + g * CPG
            pltpu.sync_copy(edge_hbm.at[0, pl.ds(row0, CPG), :], idx_src)
            pltpu.sync_copy(edge_hbm.at[1, pl.ds(row0, CPG), :], idx_dst)

            for j in range(LA):
                g_issue(j, j)
            g_wait(0, 0)
            s_issue(0, 0)
            g_issue(LA, LA)

            def chunk(j, carry2):
                b = j % NB
                g_wait(j, b)
                s_issue(j, b)
                s_drain()
                g_issue(j + LA, (j + LA) % NB)
                return carry2

            lax.fori_loop(1, CPG - LA, chunk, carry)

            for j in range(CPG - LA, CPG):
                b = j % NB
                g_wait(j, b)
                s_issue(j, b)
                s_drain()
            s_drain()

            if with_counts:
                def cdrain(j, carry2):
                    pltpu.make_async_copy(vec1.at[pl.ds(0, CB)],
                                          cnt_sh.at[idx_dst.at[0]],
                                          csem).wait()
                    return carry2
                lax.fori_loop(0, CPG, cdrain, carry)
            return carry

        lax.fori_loop(0, GROUPS, group, 0)
        plsc.subcore_barrier()

        pltpu.sync_copy(
            acc_sh.at[pl.ds(s * ROWS_PER_TILE, ROWS_PER_TILE), :],
            out_hbm.at[c, pl.ds(s * ROWS_PER_TILE, ROWS_PER_TILE), :])
        if with_counts:
            pltpu.sync_copy(
                cnt_sh.at[pl.ds(s * ROWS_PER_TILE, ROWS_PER_TILE)],
                cnt_hbm.at[c, pl.ds(s * ROWS_PER_TILE, ROWS_PER_TILE)])

    return agg


_DN = (((1,), (1,)), ((), ()))


def _tc_self_body(x_ref, w_ref, b_ref, o_ref):
    o_ref[...] = lax.dot_general(
        x_ref[...], w_ref[...], _DN,
        preferred_element_type=jnp.float32) + b_ref[...]


def _tc_mix1_body(P_ref, C_ref, s_ref, wl_ref, h_ref, ic_ref):
    acc = P_ref[0] + P_ref[1]
    cnt = C_ref[0] + C_ref[1]
    inv = 1.0 / jnp.maximum(cnt, 1.0)
    h = lax.dot_general(acc * inv, wl_ref[...], _DN,
                        preferred_element_type=jnp.float32) + s_ref[...]
    h_ref[...] = jnp.maximum(h, 0.0)
    ic_ref[...] = inv


def _tc_mix2_body(P_ref, s_ref, ic_ref, wl_ref, out_ref):
    acc = P_ref[0] + P_ref[1]
    out_ref[...] = lax.dot_general(
        acc * ic_ref[...], wl_ref[...], _DN,
        preferred_element_type=jnp.float32) + s_ref[...]


def _tc_self(x, w, b):
    return pl.pallas_call(
        _tc_self_body,
        grid=(N // R,),
        in_specs=[
            pl.BlockSpec((R, D), lambda i: (i, 0)),
            pl.BlockSpec((D, D), lambda i: (0, 0)),
            pl.BlockSpec((1, D), lambda i: (0, 0)),
        ],
        out_specs=pl.BlockSpec((R, D), lambda i: (i, 0)),
        out_shape=jax.ShapeDtypeStruct((N, D), jnp.float32),
    )(x, w, b)


def _tc_mix1(P, C3, s, wl):
    return pl.pallas_call(
        _tc_mix1_body,
        grid=(N // R,),
        in_specs=[
            pl.BlockSpec((NC, R, D), lambda i: (0, i, 0)),
            pl.BlockSpec((NC, R, 1), lambda i: (0, i, 0)),
            pl.BlockSpec((R, D), lambda i: (i, 0)),
            pl.BlockSpec((D, D), lambda i: (0, 0)),
        ],
        out_specs=[
            pl.BlockSpec((R, D), lambda i: (i, 0)),
            pl.BlockSpec((R, 1), lambda i: (i, 0)),
        ],
        out_shape=[
            jax.ShapeDtypeStruct((N, D), jnp.float32),
            jax.ShapeDtypeStruct((N, 1), jnp.float32),
        ],
    )(P, C3, s, wl)


def _tc_mix2(P, s, ic, wl):
    return pl.pallas_call(
        _tc_mix2_body,
        grid=(N // R,),
        in_specs=[
            pl.BlockSpec((NC, R, D), lambda i: (0, i, 0)),
            pl.BlockSpec((R, D), lambda i: (i, 0)),
            pl.BlockSpec((R, 1), lambda i: (i, 0)),
            pl.BlockSpec((D, D), lambda i: (0, 0)),
        ],
        out_specs=pl.BlockSpec((R, D), lambda i: (i, 0)),
        out_shape=jax.ShapeDtypeStruct((N, D), jnp.float32),
    )(P, s, ic, wl)


def kernel(x, edge_index, W1_l, b1_l, W1_r, W2_l, b2_l, W2_r):
    edge3 = edge_index.reshape(2, E // CB, CB)
    s1 = _tc_self(x, W1_r, b1_l.reshape(1, D))
    P1, C1 = _make_sc_agg(True)(x, edge3)
    h, ic = _tc_mix1(P1, C1.reshape(NC, NP, 1), s1, W1_l)
    s2 = _tc_self(h, W2_r, b2_l.reshape(1, D))
    P2, = _make_sc_agg(False)(h, edge3)
    return _tc_mix2(P2, s2, ic, W2_l)
